# baseline (speedup 1.0000x reference)
import functools

import jax
import jax.numpy as jnp
from jax import lax
from jax.experimental import pallas as pl
from jax.experimental.pallas import tpu as pltpu
from jax.experimental.pallas import tpu_sc as plsc

_N = 10000
_E = 320000
_D = 128
_H = 128

_NC = 2
_NS = 16
_NW = _NC * _NS
_CH = 128
_EPAD = 327680
_ECH = _EPAD // _CH
_NCH = _ECH // _NW
_NBUF = 2
_HCH = _NCH // 2
_HGRP = _HCH // _NBUF
_NBUF2 = 8
_NGRP2 = _NCH // _NBUF2
_NP = 10240
_RPT = _NP // _NS


_BN = 2000


def _mm_body(x_ref, wl_ref, y_ref):
    y_ref[...] = jnp.dot(x_ref[...], wl_ref[...],
                         preferred_element_type=jnp.float32)


def _tc_matmul(x, w1l_t):
    return pl.pallas_call(
        _mm_body,
        grid=(_N // _BN,),
        in_specs=[
            pl.BlockSpec((_BN, _D), lambda i: (i, 0)),
            pl.BlockSpec((_D, _H), lambda i: (0, 0)),
        ],
        out_specs=pl.BlockSpec((_BN, _H), lambda i: (i, 0)),
        out_shape=jax.ShapeDtypeStruct((_N, _H), jnp.float32),
    )(x, w1l_t)


_SC_MESH = plsc.VectorSubcoreMesh(core_axis_name="c", subcore_axis_name="s")


@functools.partial(
    pl.kernel,
    out_type=[
        jax.ShapeDtypeStruct((_NC, _NP, _H), jnp.float32),
        jax.ShapeDtypeStruct((_NC, _NP), jnp.float32),
    ],
    mesh=_SC_MESH,
    scratch_types=[
        pltpu.VMEM((_HCH, _CH), jnp.int32),
        pltpu.VMEM((_HCH, _CH), jnp.int32),
        pltpu.VMEM((_CH,), jnp.float32),
        pltpu.VMEM_SHARED((_NP, _H), jnp.float32),
        pltpu.VMEM_SHARED((_NP,), jnp.float32),
    ]
    + [pltpu.VMEM((_CH, _H), jnp.float32)] * _NBUF
    + [pltpu.SemaphoreType.DMA] * (2 * _NBUF),
)
def _sc_agg_rows(y_hbm, src_hbm, dst_hbm, zrow_hbm, zvec_hbm,
                 agg_out, deg_out,
                 si, di, ones_v, acc_sh, deg_sh, *bufsems):
    rows = list(bufsems[:_NBUF])
    gsem = list(bufsems[_NBUF:2 * _NBUF])
    ssem = list(bufsems[2 * _NBUF:])
    c = lax.axis_index("c")
    s = lax.axis_index("s")
    w = c * _NS + s
    r0 = pl.multiple_of(s * _RPT, 8)
    pltpu.sync_copy(zrow_hbm, acc_sh.at[pl.ds(r0, _RPT)])
    pltpu.sync_copy(zvec_hbm, deg_sh.at[pl.ds(r0, _RPT)])
    for i in range(_CH // 16):
        ones_v[pl.ds(i * 16, 16)] = jnp.ones((16,), jnp.float32)
    plsc.subcore_barrier()

    row0 = pl.multiple_of(w * _NCH, 8)
    for half in range(2):
        hrow = pl.multiple_of(row0 + half * _HCH, 8)
        pltpu.sync_copy(src_hbm.at[pl.ds(hrow, _HCH)], si)
        pltpu.sync_copy(dst_hbm.at[pl.ds(hrow, _HCH)], di)

        for b in range(_NBUF):
            pltpu.async_copy(y_hbm.at[si.at[b]], rows[b], gsem[b])

        def group(g, carry):
            for b in range(_NBUF):
                k = g * _NBUF + b
                pltpu.make_async_copy(
                    zrow_hbm.at[pl.ds(0, _CH)], rows[b], gsem[b]).wait()
                pltpu.async_copy(
                    rows[b], acc_sh.at[di.at[k]], ssem[b], add=True)
                pltpu.async_copy(
                    ones_v, deg_sh.at[di.at[k]], ssem[b], add=True)

                @pl.when(g < _HGRP - 1)
                def _():
                    pltpu.make_async_copy(
                        zrow_hbm.at[pl.ds(0, _CH)], rows[b], ssem[b]).wait()
                    pltpu.make_async_copy(
                        zvec_hbm.at[pl.ds(0, _CH)], ones_v, ssem[b]).wait()
                    pltpu.async_copy(
                        y_hbm.at[si.at[k + _NBUF]], rows[b], gsem[b])
            return carry

        lax.fori_loop(0, _HGRP, group, 0)
        for b in range(_NBUF):
            pltpu.make_async_copy(
                zrow_hbm.at[pl.ds(0, _CH)], rows[b], ssem[b]).wait()
            pltpu.make_async_copy(
                zvec_hbm.at[pl.ds(0, _CH)], ones_v, ssem[b]).wait()

    plsc.subcore_barrier()
    pltpu.sync_copy(acc_sh.at[pl.ds(r0, _RPT)], agg_out.at[c, pl.ds(r0, _RPT)])
    pltpu.sync_copy(deg_sh.at[pl.ds(r0, _RPT)], deg_out.at[c, pl.ds(r0, _RPT)])


def _mid_body(agg_ref, deg_ref, x_ref, w1r_ref, b1_ref, w2l_ref, w2r_ref,
              p_ref, q_ref, inv_ref):
    dsum = deg_ref[0, :, 0] + deg_ref[1, :, 0]
    inv = 1.0 / jnp.maximum(dsum, 1.0)
    mean = (agg_ref[0] + agg_ref[1]) * inv[:, None]
    z = jnp.dot(x_ref[...], w1r_ref[...], preferred_element_type=jnp.float32)
    h = jnp.maximum(mean + b1_ref[...] + z, 0.0)
    p_ref[...] = jnp.sum(h * w2l_ref[...], axis=1, keepdims=True)
    q_ref[...] = jnp.sum(h * w2r_ref[...], axis=1, keepdims=True)
    inv_ref[...] = inv[:, None]


def _tc_mid(agg, deg, x, w1r_t, b1l, w2l, w2r):
    return pl.pallas_call(
        _mid_body,
        grid=(_N // _BN,),
        in_specs=[
            pl.BlockSpec((_NC, _BN, _H), lambda i: (0, i, 0)),
            pl.BlockSpec((_NC, _BN, 1), lambda i: (0, i, 0)),
            pl.BlockSpec((_BN, _D), lambda i: (i, 0)),
            pl.BlockSpec((_D, _H), lambda i: (0, 0)),
            pl.BlockSpec((1, _H), lambda i: (0, 0)),
            pl.BlockSpec((1, _H), lambda i: (0, 0)),
            pl.BlockSpec((1, _H), lambda i: (0, 0)),
        ],
        out_specs=[
            pl.BlockSpec((_BN, 1), lambda i: (i, 0)),
            pl.BlockSpec((_BN, 1), lambda i: (i, 0)),
            pl.BlockSpec((_BN, 1), lambda i: (i, 0)),
        ],
        out_shape=[
            jax.ShapeDtypeStruct((_N, 1), jnp.float32),
            jax.ShapeDtypeStruct((_N, 1), jnp.float32),
            jax.ShapeDtypeStruct((_N, 1), jnp.float32),
        ],
    )(agg, deg, x, w1r_t, b1l, w2l, w2r)


@functools.partial(
    pl.kernel,
    out_type=jax.ShapeDtypeStruct((_NC, _NP), jnp.float32),
    mesh=_SC_MESH,
    scratch_types=[
        pltpu.VMEM((_NCH, _CH), jnp.int32),
        pltpu.VMEM((_NCH, _CH), jnp.int32),
        pltpu.VMEM_SHARED((_NP,), jnp.float32),
    ]
    + [pltpu.VMEM((_CH,), jnp.float32)] * _NBUF2
    + [pltpu.SemaphoreType.DMA] * (2 * _NBUF2),
)
def _sc_agg_scalar(p_hbm, src_hbm, dst_hbm, zvec_hbm, out,
                   si, di, acc_sh, *bufsems):
    vals = list(bufsems[:_NBUF2])
    gsem = list(bufsems[_NBUF2:2 * _NBUF2])
    ssem = list(bufsems[2 * _NBUF2:])
    c = lax.axis_index("c")
    s = lax.axis_index("s")
    w = c * _NS + s
    r0 = pl.multiple_of(s * _RPT, 8)
    pltpu.sync_copy(zvec_hbm, acc_sh.at[pl.ds(r0, _RPT)])
    row0 = pl.multiple_of(w * _NCH, 8)
    pltpu.sync_copy(src_hbm.at[pl.ds(row0, _NCH)], si)
    pltpu.sync_copy(dst_hbm.at[pl.ds(row0, _NCH)], di)
    plsc.subcore_barrier()

    for b in range(_NBUF2):
        pltpu.async_copy(p_hbm.at[si.at[b]], vals[b], gsem[b])

    def group(g, carry):
        for b in range(_NBUF2):
            k = g * _NBUF2 + b
            pltpu.make_async_copy(
                zvec_hbm.at[pl.ds(0, _CH)], vals[b], gsem[b]).wait()
            pltpu.async_copy(vals[b], acc_sh.at[di.at[k]], ssem[b], add=True)

            @pl.when(g < _NGRP2 - 1)
            def _():
                pltpu.make_async_copy(
                    zvec_hbm.at[pl.ds(0, _CH)], vals[b], ssem[b]).wait()
                pltpu.async_copy(p_hbm.at[si.at[k + _NBUF2]], vals[b], gsem[b])
        return carry

    lax.fori_loop(0, _NGRP2, group, 0)
    for b in range(_NBUF2):
        pltpu.make_async_copy(
            zvec_hbm.at[pl.ds(0, _CH)], vals[b], ssem[b]).wait()
    plsc.subcore_barrier()
    pltpu.sync_copy(acc_sh.at[pl.ds(r0, _RPT)], out.at[c, pl.ds(r0, _RPT)])


def _fin_body(a2_ref, inv_ref, q_ref, b2_ref, out_ref):
    a2 = a2_ref[0, :_N] + a2_ref[1, :_N]
    sval = a2 * inv_ref[:, 0] + b2_ref[0, 0] + q_ref[:, 0]
    out_ref[...] = jax.nn.sigmoid(sval)


def _tc_final(a2, inv, q, b2l):
    return pl.pallas_call(
        _fin_body,
        out_shape=jax.ShapeDtypeStruct((_N,), jnp.float32),
    )(a2, inv, q, b2l)


def kernel(x, edge_index, W1l, b1l, W1r, W2l, b2l, W2r):
    npad = _EPAD - _E
    pad_src = jnp.arange(npad, dtype=jnp.int32) % _N
    pad_dst = _N + jnp.arange(npad, dtype=jnp.int32) % (_NP - _N)
    src2 = jnp.concatenate([edge_index[0], pad_src]).reshape(_ECH, _CH)
    dst2 = jnp.concatenate([edge_index[1], pad_dst]).reshape(_ECH, _CH)
    zrow = jnp.zeros((_RPT, _H), jnp.float32)
    zvec = jnp.zeros((_RPT,), jnp.float32)

    y1 = _tc_matmul(x, W1l.T)
    agg, deg = _sc_agg_rows(y1, src2, dst2, zrow, zvec)
    p, q, inv = _tc_mid(agg, deg.reshape(_NC, _NP, 1), x, W1r.T,
                        b1l.reshape(1, _H), W2l, W2r)
    a2 = _sc_agg_scalar(p.reshape(_N), src2, dst2, zvec)
    out = _tc_final(a2, inv, q, b2l.reshape(1, 1))
    return out


# speedup vs baseline: 1.1193x; 1.1193x over previous
import functools

import jax
import jax.numpy as jnp
from jax import lax
from jax.experimental import pallas as pl
from jax.experimental.pallas import tpu as pltpu
from jax.experimental.pallas import tpu_sc as plsc

_N = 10000
_E = 320000
_D = 128
_H = 128

_NC = 2
_NS = 16
_NW = _NC * _NS
_CH = 128
_EPAD = 327680
_ECH = _EPAD // _CH
_NCH = _ECH // _NW
_NBUF = 2
_HCH = _NCH // 2
_HGRP = _HCH // _NBUF
_NBUF2 = 8
_NGRP2 = _NCH // _NBUF2
_NP = 10240
_RPT = _NP // _NS


_BN = 2000


def _mm_body(x_ref, wl_ref, y_ref):
    y_ref[...] = jnp.dot(x_ref[...], wl_ref[...],
                         preferred_element_type=jnp.float32)


def _tc_matmul(x, w1l_t):
    return pl.pallas_call(
        _mm_body,
        grid=(_N // _BN,),
        in_specs=[
            pl.BlockSpec((_BN, _D), lambda i: (i, 0)),
            pl.BlockSpec((_D, _H), lambda i: (0, 0)),
        ],
        out_specs=pl.BlockSpec((_BN, _H), lambda i: (i, 0)),
        out_shape=jax.ShapeDtypeStruct((_N, _H), jnp.float32),
    )(x, w1l_t)


_SC_MESH = plsc.VectorSubcoreMesh(core_axis_name="c", subcore_axis_name="s")


@functools.partial(
    pl.kernel,
    out_type=[
        jax.ShapeDtypeStruct((_NC, _NP, _H), jnp.float32),
        jax.ShapeDtypeStruct((_NC, _NP), jnp.float32),
    ],
    mesh=_SC_MESH,
    scratch_types=[
        pltpu.VMEM((_HCH, _CH), jnp.int32),
        pltpu.VMEM((_HCH, _CH), jnp.int32),
        pltpu.VMEM((_CH,), jnp.float32),
        pltpu.VMEM_SHARED((_NP, _H), jnp.float32),
        pltpu.VMEM_SHARED((_NP,), jnp.float32),
    ]
    + [pltpu.VMEM((_CH, _H), jnp.float32)] * _NBUF
    + [pltpu.SemaphoreType.DMA] * (2 * _NBUF),
)
def _sc_agg_rows(y_hbm, src_hbm, dst_hbm, zrow_hbm, zvec_hbm,
                 agg_out, deg_out,
                 si, di, ones_v, acc_sh, deg_sh, *bufsems):
    rows = list(bufsems[:_NBUF])
    gsem = list(bufsems[_NBUF:2 * _NBUF])
    ssem = list(bufsems[2 * _NBUF:])
    c = lax.axis_index("c")
    s = lax.axis_index("s")
    w = c * _NS + s
    r0 = pl.multiple_of(s * _RPT, 8)
    pltpu.sync_copy(zrow_hbm, acc_sh.at[pl.ds(r0, _RPT)])
    pltpu.sync_copy(zvec_hbm, deg_sh.at[pl.ds(r0, _RPT)])
    for i in range(_CH // 16):
        ones_v[pl.ds(i * 16, 16)] = jnp.ones((16,), jnp.float32)
    plsc.subcore_barrier()

    row0 = pl.multiple_of(w * _NCH, 8)
    for half in range(2):
        hrow = pl.multiple_of(row0 + half * _HCH, 8)
        pltpu.sync_copy(src_hbm.at[pl.ds(hrow, _HCH)], si)
        pltpu.sync_copy(dst_hbm.at[pl.ds(hrow, _HCH)], di)

        for b in range(_NBUF):
            pltpu.async_copy(y_hbm.at[si.at[b]], rows[b], gsem[b])

        def group(g, carry):
            for b in range(_NBUF):
                k = g * _NBUF + b
                pltpu.make_async_copy(
                    zrow_hbm.at[pl.ds(0, _CH)], rows[b], gsem[b]).wait()
                pltpu.async_copy(
                    rows[b], acc_sh.at[di.at[k]], ssem[b], add=True)
                pltpu.async_copy(
                    ones_v, deg_sh.at[di.at[k]], ssem[b], add=True)

                @pl.when(g < _HGRP - 1)
                def _():
                    pltpu.make_async_copy(
                        zrow_hbm.at[pl.ds(0, _CH)], rows[b], ssem[b]).wait()
                    pltpu.make_async_copy(
                        zvec_hbm.at[pl.ds(0, _CH)], ones_v, ssem[b]).wait()
                    pltpu.async_copy(
                        y_hbm.at[si.at[k + _NBUF]], rows[b], gsem[b])
            return carry

        lax.fori_loop(0, _HGRP, group, 0)
        for b in range(_NBUF):
            pltpu.make_async_copy(
                zrow_hbm.at[pl.ds(0, _CH)], rows[b], ssem[b]).wait()
            pltpu.make_async_copy(
                zvec_hbm.at[pl.ds(0, _CH)], ones_v, ssem[b]).wait()

    plsc.subcore_barrier()
    pltpu.sync_copy(acc_sh.at[pl.ds(r0, _RPT)], agg_out.at[c, pl.ds(r0, _RPT)])
    pltpu.sync_copy(deg_sh.at[pl.ds(r0, _RPT)], deg_out.at[c, pl.ds(r0, _RPT)])


def _mid_body(agg_ref, deg_ref, x_ref, w1r_ref, b1_ref, w2l_ref, w2r_ref,
              p_ref, q_ref, inv_ref):
    dsum = deg_ref[0, :, 0] + deg_ref[1, :, 0]
    inv = 1.0 / jnp.maximum(dsum, 1.0)
    mean = (agg_ref[0] + agg_ref[1]) * inv[:, None]
    z = jnp.dot(x_ref[...], w1r_ref[...], preferred_element_type=jnp.float32)
    h = jnp.maximum(mean + b1_ref[...] + z, 0.0)
    p_ref[...] = jnp.sum(h * w2l_ref[...], axis=1, keepdims=True)
    q_ref[...] = jnp.sum(h * w2r_ref[...], axis=1, keepdims=True)
    inv_ref[...] = inv[:, None]


def _tc_mid(agg, deg, x, w1r_t, b1l, w2l, w2r):
    return pl.pallas_call(
        _mid_body,
        grid=(_N // _BN,),
        in_specs=[
            pl.BlockSpec((_NC, _BN, _H), lambda i: (0, i, 0)),
            pl.BlockSpec((_NC, _BN, 1), lambda i: (0, i, 0)),
            pl.BlockSpec((_BN, _D), lambda i: (i, 0)),
            pl.BlockSpec((_D, _H), lambda i: (0, 0)),
            pl.BlockSpec((1, _H), lambda i: (0, 0)),
            pl.BlockSpec((1, _H), lambda i: (0, 0)),
            pl.BlockSpec((1, _H), lambda i: (0, 0)),
        ],
        out_specs=[
            pl.BlockSpec((_BN, 1), lambda i: (i, 0)),
            pl.BlockSpec((_BN, 1), lambda i: (i, 0)),
            pl.BlockSpec((_BN, 1), lambda i: (i, 0)),
        ],
        out_shape=[
            jax.ShapeDtypeStruct((_N, 1), jnp.float32),
            jax.ShapeDtypeStruct((_N, 1), jnp.float32),
            jax.ShapeDtypeStruct((_N, 1), jnp.float32),
        ],
    )(agg, deg, x, w1r_t, b1l, w2l, w2r)


_GPR = _CH // 16


@functools.partial(
    pl.kernel,
    out_type=jax.ShapeDtypeStruct((_NC, _NP), jnp.float32),
    mesh=_SC_MESH,
    scratch_types=[
        pltpu.VMEM((_N,), jnp.float32),
        pltpu.VMEM((_NP,), jnp.float32),
        pltpu.VMEM((_NCH, _CH), jnp.int32),
        pltpu.VMEM((_NCH, _CH), jnp.int32),
        pltpu.VMEM((_NS, _RPT), jnp.float32),
        pltpu.VMEM((_RPT,), jnp.float32),
        pltpu.VMEM_SHARED((_NS, _NP), jnp.float32),
    ],
    compiler_params=pltpu.CompilerParams(needs_layout_passes=False),
)
def _sc_agg_scalar(p_hbm, src_hbm, dst_hbm, out,
                   p_v, acc_v, si, di, stripes_v, red_v, part_sh):
    c = lax.axis_index("c")
    s = lax.axis_index("s")
    w = c * _NS + s
    pltpu.sync_copy(p_hbm, p_v)
    row0 = pl.multiple_of(w * _NCH, 8)
    pltpu.sync_copy(src_hbm.at[pl.ds(row0, _NCH)], si)
    pltpu.sync_copy(dst_hbm.at[pl.ds(row0, _NCH)], di)

    zv = jnp.zeros((16,), jnp.float32)
    for i in range(_NP // 16):
        acc_v[pl.ds(i * 16, 16)] = zv

    for k in range(_NCH):
        for jj in range(_GPR):
            sv = si[k, pl.ds(jj * 16, 16)]
            dv = di[k, pl.ds(jj * 16, 16)]
            vals = plsc.load_gather(p_v, [sv])
            plsc.addupdate_scatter(acc_v, [dv], vals)

    pltpu.sync_copy(acc_v, part_sh.at[s])
    plsc.subcore_barrier()
    r0 = pl.multiple_of(s * _RPT, 8)
    for t in range(_NS):
        pltpu.sync_copy(part_sh.at[t, pl.ds(r0, _RPT)], stripes_v.at[t])

    for g in range(_RPT // 16):
        acc16 = stripes_v[0, pl.ds(g * 16, 16)]
        for t in range(1, _NS):
            acc16 = acc16 + stripes_v[t, pl.ds(g * 16, 16)]
        red_v[pl.ds(g * 16, 16)] = acc16
    pltpu.sync_copy(red_v, out.at[c, pl.ds(r0, _RPT)])


def _fin_body(a2_ref, inv_ref, q_ref, b2_ref, out_ref):
    a2 = a2_ref[0, :_N] + a2_ref[1, :_N]
    sval = a2 * inv_ref[:, 0] + b2_ref[0, 0] + q_ref[:, 0]
    out_ref[...] = jax.nn.sigmoid(sval)


def _tc_final(a2, inv, q, b2l):
    return pl.pallas_call(
        _fin_body,
        out_shape=jax.ShapeDtypeStruct((_N,), jnp.float32),
    )(a2, inv, q, b2l)


def kernel(x, edge_index, W1l, b1l, W1r, W2l, b2l, W2r):
    npad = _EPAD - _E
    pad_src = jnp.arange(npad, dtype=jnp.int32) % _N
    pad_dst = _N + jnp.arange(npad, dtype=jnp.int32) % (_NP - _N)
    src2 = jnp.concatenate([edge_index[0], pad_src]).reshape(_ECH, _CH)
    dst2 = jnp.concatenate([edge_index[1], pad_dst]).reshape(_ECH, _CH)
    zrow = jnp.zeros((_RPT, _H), jnp.float32)
    zvec = jnp.zeros((_RPT,), jnp.float32)

    y1 = _tc_matmul(x, W1l.T)
    agg, deg = _sc_agg_rows(y1, src2, dst2, zrow, zvec)
    p, q, inv = _tc_mid(agg, deg.reshape(_NC, _NP, 1), x, W1r.T,
                        b1l.reshape(1, _H), W2l, W2r)
    a2 = _sc_agg_scalar(p.reshape(_N), src2, dst2)
    out = _tc_final(a2, inv, q, b2l.reshape(1, 1))
    return out
